# baseline (speedup 1.0000x reference)
import functools

import jax
import jax.numpy as jnp
from jax import lax
from jax.experimental import pallas as pl
from jax.experimental.pallas import tpu as pltpu
from jax.experimental.pallas import tpu_sc as plsc


def _make_bcast_kernel(batch, seq, hidden, dtype):
    info = plsc.get_sparse_core_info()
    nw = info.num_cores * info.num_subcores
    total_rows = batch * seq
    rows_per_w = total_rows // nw
    chunks_per_seq = seq // rows_per_w

    mesh = plsc.VectorSubcoreMesh(core_axis_name="c", subcore_axis_name="s")

    @functools.partial(
        pl.kernel,
        mesh=mesh,
        out_type=jax.ShapeDtypeStruct((total_rows, hidden), dtype),
        scratch_types=[pltpu.SemaphoreType.DMA],
    )
    def k(w_hbm, out_hbm, sem):
        wid = lax.axis_index("s") * info.num_cores + lax.axis_index("c")
        soff = (wid % chunks_per_seq) * rows_per_w
        pltpu.async_copy(
            w_hbm.at[pl.ds(soff, rows_per_w)],
            out_hbm.at[pl.ds(wid * rows_per_w, rows_per_w)],
            sem,
        ).wait()

    return k


def kernel(input_ids, pos_emb_weight):
    batch, seq = input_ids.shape
    hidden = pos_emb_weight.shape[1]
    k = _make_bcast_kernel(batch, seq, hidden, pos_emb_weight.dtype)
    out = k(pos_emb_weight[:seq])
    return out.reshape(batch, seq, hidden)


# speedup vs baseline: 44.4368x; 44.4368x over previous
import functools

import jax
import jax.numpy as jnp
from jax import lax
from jax.experimental import pallas as pl
from jax.experimental.pallas import tpu as pltpu
from jax.experimental.pallas import tpu_sc as plsc

_CHUNK = 16


def _make_bcast_kernel(batch, seq, hidden, dtype):
    info = plsc.get_sparse_core_info()
    nw = info.num_cores * info.num_subcores
    rows_per_w = seq // nw
    n_chunks = rows_per_w // _CHUNK

    mesh = plsc.VectorSubcoreMesh(core_axis_name="c", subcore_axis_name="s")

    @functools.partial(
        pl.kernel,
        mesh=mesh,
        out_type=jax.ShapeDtypeStruct((batch, seq, hidden), dtype),
        scratch_types=[
            pltpu.VMEM((2, _CHUNK, hidden), dtype),
            pltpu.SemaphoreType.DMA((2,)),
            pltpu.SemaphoreType.DMA,
        ],
    )
    def k(w_hbm, out_hbm, buf, load_sem, store_sem):
        wid = lax.axis_index("s") * info.num_cores + lax.axis_index("c")
        base = wid * rows_per_w

        def load(c, slot):
            return pltpu.make_async_copy(
                w_hbm.at[pl.ds(base + c * _CHUNK, _CHUNK)],
                buf.at[slot],
                load_sem.at[slot],
            )

        load(0, 0).start()

        def body(c, _):
            slot = lax.rem(c, 2)
            load(c, slot).wait()

            @pl.when(c + 1 < n_chunks)
            def _():
                load(c + 1, 1 - slot).start()

            copies = [
                pltpu.make_async_copy(
                    buf.at[slot],
                    out_hbm.at[b, pl.ds(base + c * _CHUNK, _CHUNK)],
                    store_sem,
                )
                for b in range(batch)
            ]
            for cp in copies:
                cp.start()
            for cp in copies:
                cp.wait()
            return 0

        lax.fori_loop(0, n_chunks, body, 0)

    return k


def kernel(input_ids, pos_emb_weight):
    batch, seq = input_ids.shape
    hidden = pos_emb_weight.shape[1]
    k = _make_bcast_kernel(batch, seq, hidden, pos_emb_weight.dtype)
    return k(pos_emb_weight[:seq])
